# baseline (speedup 1.0000x reference)
import functools

import jax
import jax.numpy as jnp
from jax.experimental import pallas as pl

NUM_EXPERTS = 64
TOP_K = 8
HIDDEN_SIZE = 2048
NUM_TOKENS = 16384

BT = 1024


def _process(h, p):
    hn = jnp.maximum(jnp.sqrt(jnp.sum(h * h, axis=1, keepdims=True)), 1e-12)
    pn = jnp.maximum(jnp.sqrt(jnp.sum(p * p, axis=1, keepdims=True)), 1e-12)
    hb = (h / hn).astype(jnp.bfloat16)
    pb = (p / pn).astype(jnp.bfloat16)
    lt = jax.lax.dot_general(
        pb, hb, (((1,), (1,)), ((), ())), preferred_element_type=jnp.float32
    )
    m = jnp.max(lt, axis=0, keepdims=True)
    e = jnp.exp(lt - m)
    probs = e / jnp.sum(e, axis=0, keepdims=True)

    iota = jax.lax.broadcasted_iota(jnp.int32, probs.shape, 0).astype(jnp.float32)
    row8 = jax.lax.broadcasted_iota(jnp.int32, (TOP_K, probs.shape[1]), 0).astype(
        jnp.float32
    )
    x = probs
    acc_w = jnp.zeros((TOP_K, probs.shape[1]), jnp.float32)
    acc_i = jnp.zeros((TOP_K, probs.shape[1]), jnp.float32)
    for k in range(TOP_K):
        mk = jnp.max(x, axis=0, keepdims=True)
        imf = jnp.min(
            jnp.where(x == mk, iota, float(NUM_EXPERTS)), axis=0, keepdims=True
        )
        acc_w = acc_w + jnp.where(row8 == float(k), mk, 0.0)
        acc_i = acc_i + jnp.where(row8 == float(k), imf, 0.0)
        x = jnp.where(iota == imf, -1.0, x)
    return acc_w.T, acc_i.T.astype(jnp.int32)


def _router_body(h1_ref, h2_ref, p_ref, w_ref, i_ref):
    p = p_ref[...]
    w1, i1 = _process(h1_ref[...], p)
    w_ref[0:BT, :] = w1
    i_ref[0:BT, :] = i1
    w2, i2 = _process(h2_ref[...], p)
    w_ref[BT : 2 * BT, :] = w2
    i_ref[BT : 2 * BT, :] = i2


@jax.jit
def kernel(hidden_states, proto):
    grid = (NUM_TOKENS // (2 * BT),)
    return pl.pallas_call(
        _router_body,
        grid=grid,
        in_specs=[
            pl.BlockSpec((BT, HIDDEN_SIZE), lambda t: (2 * t, 0)),
            pl.BlockSpec((BT, HIDDEN_SIZE), lambda t: (2 * t + 1, 0)),
            pl.BlockSpec((NUM_EXPERTS, HIDDEN_SIZE), lambda t: (0, 0)),
        ],
        out_specs=[
            pl.BlockSpec((2 * BT, TOP_K), lambda t: (t, 0)),
            pl.BlockSpec((2 * BT, TOP_K), lambda t: (t, 0)),
        ],
        out_shape=[
            jax.ShapeDtypeStruct((NUM_TOKENS, TOP_K), jnp.float32),
            jax.ShapeDtypeStruct((NUM_TOKENS, TOP_K), jnp.int32),
        ],
    )(hidden_states, hidden_states, proto)


# speedup vs baseline: 1.0144x; 1.0144x over previous
import jax
import jax.numpy as jnp
from jax.experimental import pallas as pl

NUM_EXPERTS = 64
TOP_K = 8
HIDDEN_SIZE = 2048
NUM_TOKENS = 16384

BT = 1024


def _process(h, p):
    hn = jnp.maximum(jnp.sqrt(jnp.sum(h * h, axis=1, keepdims=True)), 1e-12)
    pn = jnp.maximum(jnp.sqrt(jnp.sum(p * p, axis=1, keepdims=True)), 1e-12)
    hb = (h / hn).astype(jnp.bfloat16)
    pb = (p / pn).astype(jnp.bfloat16)
    logits = jax.lax.dot_general(
        hb, pb, (((1,), (1,)), ((), ())), preferred_element_type=jnp.float32
    )
    lt = logits.T
    m = jnp.max(lt, axis=0, keepdims=True)
    e = jnp.exp(lt - m)
    probs = e / jnp.sum(e, axis=0, keepdims=True)

    iota = jax.lax.broadcasted_iota(jnp.int32, probs.shape, 0).astype(jnp.float32)
    row8 = jax.lax.broadcasted_iota(jnp.int32, (TOP_K, probs.shape[1]), 0).astype(
        jnp.float32
    )
    x = probs
    acc_w = jnp.zeros((TOP_K, probs.shape[1]), jnp.float32)
    acc_i = jnp.zeros((TOP_K, probs.shape[1]), jnp.float32)
    for k in range(TOP_K):
        mk = jnp.max(x, axis=0, keepdims=True)
        imf = jnp.min(
            jnp.where(x == mk, iota, float(NUM_EXPERTS)), axis=0, keepdims=True
        )
        acc_w = acc_w + jnp.where(row8 == float(k), mk, 0.0)
        acc_i = acc_i + jnp.where(row8 == float(k), imf, 0.0)
        x = jnp.where(iota == imf, -1.0, x)
    return acc_w.T, acc_i.T.astype(jnp.int32)


def _router_body(h1_ref, h2_ref, p_ref, w_ref, i_ref):
    p = p_ref[...]
    w1, i1 = _process(h1_ref[...], p)
    w_ref[0:BT, :] = w1
    i_ref[0:BT, :] = i1
    w2, i2 = _process(h2_ref[...], p)
    w_ref[BT : 2 * BT, :] = w2
    i_ref[BT : 2 * BT, :] = i2


@jax.jit
def kernel(hidden_states, proto):
    grid = (NUM_TOKENS // (2 * BT),)
    return pl.pallas_call(
        _router_body,
        grid=grid,
        in_specs=[
            pl.BlockSpec((BT, HIDDEN_SIZE), lambda t: (2 * t, 0)),
            pl.BlockSpec((BT, HIDDEN_SIZE), lambda t: (2 * t + 1, 0)),
            pl.BlockSpec((NUM_EXPERTS, HIDDEN_SIZE), lambda t: (0, 0)),
        ],
        out_specs=[
            pl.BlockSpec((2 * BT, TOP_K), lambda t: (t, 0)),
            pl.BlockSpec((2 * BT, TOP_K), lambda t: (t, 0)),
        ],
        out_shape=[
            jax.ShapeDtypeStruct((NUM_TOKENS, TOP_K), jnp.float32),
            jax.ShapeDtypeStruct((NUM_TOKENS, TOP_K), jnp.int32),
        ],
    )(hidden_states, hidden_states, proto)
